# baseline (speedup 1.0000x reference)
import functools

import jax
import jax.numpy as jnp
from jax import lax
from jax.experimental import pallas as pl
from jax.experimental.pallas import tpu as pltpu
from jax.experimental.pallas import tpu_sc as plsc

_FEAT = 2048
_NCLS = 1000
_BATCH = 16384
_NC = 2
_NS = 16
_NW = _NC * _NS
_BPW = _BATCH // _NW
_NSEM = 16


def _sc_gather(table, idx):
    mesh = plsc.VectorSubcoreMesh(core_axis_name="c", subcore_axis_name="s")

    @functools.partial(
        pl.kernel,
        out_type=jax.ShapeDtypeStruct((_BATCH * _FEAT,), jnp.float32),
        mesh=mesh,
        scratch_types=[
            pltpu.VMEM((_BPW,), jnp.int32),
            pltpu.VMEM_SHARED((_NCLS * _FEAT,), jnp.float32),
        ]
        + [pltpu.SemaphoreType.DMA for _ in range(_NSEM)],
    )
    def k(table_hbm, idx_hbm, out_hbm, idx_v, shared, *sems):
        sid = lax.axis_index("s")
        wid = sid * _NC + lax.axis_index("c")
        base = wid * _BPW
        pltpu.sync_copy(idx_hbm.at[pl.ds(base, _BPW)], idx_v)

        rows_lo = 62
        off = pl.multiple_of(sid * rows_lo * _FEAT, 8)

        @pl.when(sid < 15)
        def _():
            pltpu.sync_copy(
                table_hbm.at[pl.ds(off, rows_lo * _FEAT)],
                shared.at[pl.ds(off, rows_lo * _FEAT)],
            )

        @pl.when(sid == 15)
        def _():
            pltpu.sync_copy(
                table_hbm.at[pl.ds(15 * rows_lo * _FEAT, 70 * _FEAT)],
                shared.at[pl.ds(15 * rows_lo * _FEAT, 70 * _FEAT)],
            )

        plsc.subcore_barrier()

        def row_copy(i, r, sem):
            pltpu.async_copy(
                shared.at[pl.ds(r * _FEAT, _FEAT)],
                out_hbm.at[pl.ds((base + i) * _FEAT, _FEAT)],
                sem,
            )

        def row_wait(sem):
            pltpu.make_async_copy(
                shared.at[pl.ds(0, _FEAT)],
                out_hbm.at[pl.ds(0, _FEAT)],
                sem,
            ).wait()

        @pl.loop(0, _BPW // 16)
        def _(g):
            v = idx_v[pl.ds(g * 16, 16)]
            for j in range(16):
                r = v[j]

                @pl.when(g > 0)
                def _():
                    row_wait(sems[j])

                row_copy(g * 16 + j, r, sems[j])

        for j in range(_NSEM):
            row_wait(sems[j])

    flat = k(table.reshape(-1), idx)
    return flat.reshape(_BATCH, _FEAT)


def kernel(label, proto):
    table = proto.T
    return _sc_gather(table, label.astype(jnp.int32))


# speedup vs baseline: 2.0378x; 2.0378x over previous
import functools

import jax
import jax.numpy as jnp
from jax import lax
from jax.experimental import pallas as pl
from jax.experimental.pallas import tpu as pltpu
from jax.experimental.pallas import tpu_sc as plsc

_FEAT = 2048
_NCLS = 1000
_BATCH = 16384
_NC = 2
_NS = 16
_NW = _NC * _NS
_BPW = _BATCH // _NW
_CHUNK = 16
_NBUF = 2
_NCHUNK = _BPW // _CHUNK


def _sc_gather(table, idx):
    mesh = plsc.VectorSubcoreMesh(core_axis_name="c", subcore_axis_name="s")

    @functools.partial(
        pl.kernel,
        out_type=jax.ShapeDtypeStruct((_BATCH, _FEAT), jnp.float32),
        mesh=mesh,
        scratch_types=[
            pltpu.VMEM((_BPW,), jnp.int32),
            pltpu.VMEM((_NCHUNK, _CHUNK), jnp.int32),
        ]
        + [pltpu.VMEM((_CHUNK, _FEAT), jnp.float32) for _ in range(_NBUF)]
        + [pltpu.SemaphoreType.DMA for _ in range(2 * _NBUF)],
    )
    def k(table_hbm, idx_hbm, out_hbm, idx_v, pos2d, *bufs_and_sems):
        bufs = bufs_and_sems[:_NBUF]
        gsems = bufs_and_sems[_NBUF:2 * _NBUF]
        wsems = bufs_and_sems[2 * _NBUF:]
        wid = lax.axis_index("s") * _NC + lax.axis_index("c")
        base = wid * _BPW
        pltpu.sync_copy(idx_hbm.at[pl.ds(base, _BPW)], idx_v)

        @pl.loop(0, _NCHUNK)
        def _(g):
            pos2d[g, :] = base + g * _CHUNK + lax.iota(jnp.int32, 16)

        def start_gather(g, b):
            pltpu.async_copy(
                table_hbm.at[idx_v.at[pl.ds(g * _CHUNK, _CHUNK)]],
                bufs[b], gsems[b],
            )

        for g in range(_NBUF - 1):
            start_gather(g, g)

        @pl.loop(0, _NCHUNK, step=_NBUF)
        def _(g0):
            for b in range(_NBUF):
                g = g0 + b
                pltpu.make_async_copy(
                    table_hbm.at[idx_v.at[pl.ds(0, _CHUNK)]], bufs[b], gsems[b]
                ).wait()
                pltpu.async_copy(
                    bufs[b], out_hbm.at[pos2d.at[g]],
                    wsems[b],
                )
                gn = g + _NBUF - 1
                bn = (b + _NBUF - 1) % _NBUF

                @pl.when((gn < _NCHUNK) & (gn >= _NBUF))
                def _():
                    pltpu.make_async_copy(
                        bufs[bn],
                        out_hbm.at[pl.ds(base, _CHUNK)],
                        wsems[bn],
                    ).wait()

                @pl.when(gn < _NCHUNK)
                def _():
                    start_gather(gn, bn)

        for b in range(_NBUF):
            pltpu.make_async_copy(
                bufs[b], out_hbm.at[pl.ds(base, _CHUNK)], wsems[b]
            ).wait()

    return k(table, idx)


def kernel(label, proto):
    table = proto.T
    return _sc_gather(table, label.astype(jnp.int32))


# speedup vs baseline: 2.0990x; 1.0300x over previous
import functools

import jax
import jax.numpy as jnp
from jax import lax
from jax.experimental import pallas as pl
from jax.experimental.pallas import tpu as pltpu
from jax.experimental.pallas import tpu_sc as plsc

_FEAT = 2048
_NCLS = 1000
_BATCH = 16384
_NC = 2
_NS = 16
_NW = _NC * _NS
_BPW = _BATCH // _NW
_CHUNK = 8
_NBUF = 4
_NCHUNK = _BPW // _CHUNK


def _sc_gather(table, idx):
    mesh = plsc.VectorSubcoreMesh(core_axis_name="c", subcore_axis_name="s")

    @functools.partial(
        pl.kernel,
        out_type=jax.ShapeDtypeStruct((_BATCH, _FEAT), jnp.float32),
        mesh=mesh,
        scratch_types=[
            pltpu.VMEM((_BPW,), jnp.int32),
            pltpu.VMEM_SHARED((_NS, 2, _CHUNK, _FEAT), jnp.float32),
        ]
        + [pltpu.VMEM((_CHUNK, _FEAT), jnp.float32) for _ in range(_NBUF)]
        + [pltpu.SemaphoreType.DMA for _ in range(2 * _NBUF + 2)],
    )
    def k(table_hbm, idx_hbm, out_hbm, idx_v, shared, *bufs_and_sems):
        bufs = bufs_and_sems[:_NBUF]
        gsems = bufs_and_sems[_NBUF:2 * _NBUF]
        csems = bufs_and_sems[2 * _NBUF:3 * _NBUF]
        wsems = bufs_and_sems[3 * _NBUF:]
        assert len(wsems) == 2
        sid = lax.axis_index("s")
        wid = sid * _NC + lax.axis_index("c")
        base = wid * _BPW
        pltpu.sync_copy(idx_hbm.at[pl.ds(base, _BPW)], idx_v)

        def start_gather(g, b):
            pltpu.async_copy(
                table_hbm.at[idx_v.at[pl.ds(g * _CHUNK, _CHUNK)]],
                bufs[b], gsems[b],
            )

        for g in range(_NBUF):
            start_gather(g, g)

        @pl.loop(0, _NCHUNK, step=_NBUF)
        def _(g0):
            for b in range(_NBUF):
                g = g0 + b
                s2 = b % 2
                slot = shared.at[sid, s2]
                pltpu.make_async_copy(
                    table_hbm.at[idx_v.at[pl.ds(0, _CHUNK)]], bufs[b], gsems[b]
                ).wait()

                @pl.when(g >= 2)
                def _():
                    pltpu.make_async_copy(
                        slot, out_hbm.at[pl.ds(base, _CHUNK)], wsems[s2]
                    ).wait()

                pltpu.async_copy(bufs[b], slot, csems[b])
                pltpu.make_async_copy(bufs[b], slot, csems[b]).wait()
                pltpu.async_copy(
                    slot, out_hbm.at[pl.ds(base + g * _CHUNK, _CHUNK)],
                    wsems[s2],
                )

                @pl.when(g + _NBUF < _NCHUNK)
                def _():
                    start_gather(g + _NBUF, b)

        for s2 in range(2):
            pltpu.make_async_copy(
                shared.at[sid, s2], out_hbm.at[pl.ds(base, _CHUNK)], wsems[s2]
            ).wait()

    return k(table, idx)


def kernel(label, proto):
    table = proto.T
    return _sc_gather(table, label.astype(jnp.int32))
